# Initial kernel scaffold; baseline (speedup 1.0000x reference)
import jax
import jax.numpy as jnp
from jax.experimental import pallas as pl


def kernel(substrate_features, substrate_edge_index, vnr_features, actor_conv_W, actor_conv_b, critic_conv_W, critic_conv_b, a1_w, a1_b, a2_w, a2_b, a3_w, a3_b, c1_w, c1_b, c2_w, c2_b, c3_w, c3_b, actor_fc_w, actor_fc_b, critic_fc_w, critic_fc_b):
    raise NotImplementedError("write your pallas kernel here")


# speedup vs baseline: 4.9923x; 4.9923x over previous
import jax
import jax.numpy as jnp
from jax import lax
from jax.experimental import pallas as pl
from jax.experimental.pallas import tpu as pltpu
from jax.experimental.pallas import tpu_sc as plsc

N = 10000
D = 128
E = 320000
OUT = 3
FC_IN = (N + 3) * 3
AD = 10000

NPT = 640
NP = 16 * NPT
EPT = E // 16
B = 128
NFULL = EPT // B
TAIL = EPT - NFULL * B

ROWB = 2048
NROWB = NP // ROWB

COLB = 128
NCOLB = (AD + COLB - 1) // COLB


def _proj_body(x_ref, w_ref, b_ref, o_ref):
    o_ref[0] = (
        jnp.dot(x_ref[...], w_ref[0], preferred_element_type=jnp.float32)
        + b_ref[0]
    )


def _project(x, wcat, bias):
    return pl.pallas_call(
        _proj_body,
        grid=(2, NROWB),
        in_specs=[
            pl.BlockSpec((ROWB, D), lambda i, j: (j, 0)),
            pl.BlockSpec((1, D, 9), lambda i, j: (i, 0, 0)),
            pl.BlockSpec((1, 1, 9), lambda i, j: (i, 0, 0)),
        ],
        out_specs=pl.BlockSpec((1, ROWB, 9), lambda i, j: (i, j, 0)),
        out_shape=jax.ShapeDtypeStruct((2, NP, 9), jnp.float32),
    )(x, wcat, bias)


def _sc_body(row_h, col_h, y, tab_h, z1, ones_h, out,
             deg_sh, s1c, a1c, s2c, a2c,
             rowi, rowt, colb, oneb,
             degb, degi, disb, ybT, s1T, valb, a1T, s2T, a2T, oT):
    cid = lax.axis_index("c")
    sid = lax.axis_index("s")
    n0 = sid * NPT
    e0 = sid * EPT

    pltpu.sync_copy(z1, deg_sh.at[pl.ds(n0, NPT)])
    for r in a1c:
        pltpu.sync_copy(z1, r.at[pl.ds(n0, NPT)])
    for r in a2c:
        pltpu.sync_copy(z1, r.at[pl.ds(n0, NPT)])
    pltpu.sync_copy(ones_h, oneb)
    pltpu.sync_copy(col_h.at[pl.ds(e0, EPT)], colb)

    def load_row(j, c):
        pltpu.sync_copy(row_h.at[pl.ds(e0 + j * B, B)], rowi.at[j])
        return c
    lax.fori_loop(0, NFULL, load_row, 0)
    pltpu.sync_copy(row_h.at[pl.ds(e0 + NFULL * B, TAIL)], rowt)
    plsc.subcore_barrier()

    def deg_chunk(j, c):
        pltpu.sync_copy(oneb, deg_sh.at[rowi.at[j]], add=True)
        return c
    lax.fori_loop(0, NFULL, deg_chunk, 0)
    pltpu.sync_copy(oneb.at[pl.ds(0, TAIL)], deg_sh.at[rowt], add=True)
    plsc.subcore_barrier()

    pltpu.sync_copy(deg_sh.at[pl.ds(n0, NPT)], degb)
    for c9 in range(9):
        pltpu.sync_copy(y.at[pl.ds((cid * 9 + c9) * NP + n0, NPT)],
                        ybT.at[pl.ds(c9 * NPT, NPT)])

    def cvt_iter(i, c):
        j = i // (B // 16)
        k = i - j * (B // 16)
        degi[j, pl.ds(k * 16, 16)] = degb[pl.ds(i * 16, 16)].astype(jnp.int32)
        return c
    lax.fori_loop(0, NPT // 16, cvt_iter, 0)

    def dis_iter(j, c):
        pltpu.sync_copy(tab_h.at[degi.at[j]], disb.at[pl.ds(j * B, B)])
        return c
    lax.fori_loop(0, NPT // B, dis_iter, 0)

    def s1_iter(i, c):
        dv = disb[pl.ds(i * 16, 16)]
        for c6 in range(6):
            s1T[pl.ds(c6 * NPT + i * 16, 16)] = (
                dv * ybT[pl.ds((c6 + 3) * NPT + i * 16, 16)])
        return c
    lax.fori_loop(0, NPT // 16, s1_iter, 0)
    for c6 in range(6):
        pltpu.sync_copy(s1T.at[pl.ds(c6 * NPT, NPT)],
                        s1c[c6].at[pl.ds(n0, NPT)])
    plsc.subcore_barrier()

    def edge_pass(srcs, accs, vb):
        def chunk(j, c):
            cidx = colb.at[pl.ds(j * B, B)]
            ridx = rowi.at[j]
            for sref, aref in zip(srcs, accs):
                pltpu.sync_copy(sref.at[cidx], vb)
                pltpu.sync_copy(vb, aref.at[ridx], add=True)
            return c
        lax.fori_loop(0, NFULL, chunk, 0)
        tidx = colb.at[pl.ds(NFULL * B, TAIL)]
        vbt = vb.at[pl.ds(0, TAIL)]
        for sref, aref in zip(srcs, accs):
            pltpu.sync_copy(sref.at[tidx], vbt)
            pltpu.sync_copy(vbt, aref.at[rowt], add=True)

    edge_pass(s1c, a1c, valb)
    plsc.subcore_barrier()

    for c6 in range(6):
        pltpu.sync_copy(a1c[c6].at[pl.ds(n0, NPT)],
                        a1T.at[pl.ds(c6 * NPT, NPT)])

    def s2_iter(i, c):
        dv = disb[pl.ds(i * 16, 16)]
        for c3 in range(3):
            s2T[pl.ds(c3 * NPT + i * 16, 16)] = (
                dv * dv * a1T[pl.ds((c3 + 3) * NPT + i * 16, 16)])
        return c
    lax.fori_loop(0, NPT // 16, s2_iter, 0)
    for c3 in range(3):
        pltpu.sync_copy(s2T.at[pl.ds(c3 * NPT, NPT)],
                        s2c[c3].at[pl.ds(n0, NPT)])
    plsc.subcore_barrier()

    edge_pass(s2c, a2c, valb)
    plsc.subcore_barrier()

    for c3 in range(3):
        pltpu.sync_copy(a2c[c3].at[pl.ds(n0, NPT)],
                        a2T.at[pl.ds(c3 * NPT, NPT)])

    def out_iter(i, c):
        o16 = i * 16
        dv = disb[pl.ds(o16, 16)]
        for c3 in range(3):
            pre = (ybT[pl.ds(c3 * NPT + o16, 16)]
                   - ybT[pl.ds((c3 + 6) * NPT + o16, 16)]
                   - dv * a1T[pl.ds(c3 * NPT + o16, 16)]
                   + 2.0 * dv * a2T[pl.ds(c3 * NPT + o16, 16)])
            ex = jnp.exp(2.0 * pre)
            oT[pl.ds(c3 * NPT + o16, 16)] = 1.0 - 2.0 / (ex + 1.0)
        return c
    lax.fori_loop(0, NPT // 16, out_iter, 0)

    for c3 in range(3):
        pltpu.sync_copy(oT.at[pl.ds(c3 * NPT, NPT)],
                        out.at[pl.ds((cid * 3 + c3) * NP + n0, NPT)])


def _sc_conv(row, col, yproj):
    mesh = plsc.VectorSubcoreMesh(
        core_axis_name="c", subcore_axis_name="s", num_cores=2,
        num_subcores=16)
    z1 = jnp.zeros((NPT,), jnp.float32)
    ones_h = jnp.ones((B,), jnp.float32)
    t = jnp.arange(E + 8, dtype=jnp.float32)
    tab = jnp.where(t > 0, 1.0 / jnp.sqrt(jnp.maximum(t, 1e-12)), 0.0)
    return pl.kernel(
        _sc_body,
        out_type=jax.ShapeDtypeStruct((2 * 3 * NP,), jnp.float32),
        mesh=mesh,
        scratch_types=[
            pltpu.VMEM_SHARED((NP,), jnp.float32),
            [pltpu.VMEM_SHARED((NP,), jnp.float32)] * 6,
            [pltpu.VMEM_SHARED((NP,), jnp.float32)] * 6,
            [pltpu.VMEM_SHARED((NP,), jnp.float32)] * 3,
            [pltpu.VMEM_SHARED((NP,), jnp.float32)] * 3,
            pltpu.VMEM((NFULL, B), jnp.int32),
            pltpu.VMEM((TAIL,), jnp.int32),
            pltpu.VMEM((EPT,), jnp.int32),
            pltpu.VMEM((B,), jnp.float32),
            pltpu.VMEM((NPT,), jnp.float32),
            pltpu.VMEM((NPT // B, B), jnp.int32),
            pltpu.VMEM((NPT,), jnp.float32),
            pltpu.VMEM((9 * NPT,), jnp.float32),
            pltpu.VMEM((6 * NPT,), jnp.float32),
            pltpu.VMEM((B,), jnp.float32),
            pltpu.VMEM((6 * NPT,), jnp.float32),
            pltpu.VMEM((3 * NPT,), jnp.float32),
            pltpu.VMEM((3 * NPT,), jnp.float32),
            pltpu.VMEM((3 * NPT,), jnp.float32),
        ],
    )(row, col, yproj, tab, z1, ones_h)


def _fc_body(fa_ref, w_ref, ab_ref, fcv_ref, cw_ref, cb_ref, lo_ref, va_ref):
    lo_ref[...] = (
        jnp.dot(fa_ref[...], w_ref[...], preferred_element_type=jnp.float32)
        + ab_ref[...]
    )

    @pl.when(pl.program_id(0) == 0)
    def _():
        va_ref[...] = (
            jnp.dot(fcv_ref[...], cw_ref[...],
                    preferred_element_type=jnp.float32)
            + cb_ref[...]
        )


def _fc(fa, w, ab, fcv, cw, cb):
    return pl.pallas_call(
        _fc_body,
        grid=(NCOLB,),
        in_specs=[
            pl.BlockSpec((1, FC_IN), lambda j: (0, 0)),
            pl.BlockSpec((FC_IN, COLB), lambda j: (0, j)),
            pl.BlockSpec((1, COLB), lambda j: (0, j)),
            pl.BlockSpec((1, FC_IN), lambda j: (0, 0)),
            pl.BlockSpec((FC_IN, 1), lambda j: (0, 0)),
            pl.BlockSpec((1, 1), lambda j: (0, 0)),
        ],
        out_specs=[
            pl.BlockSpec((1, COLB), lambda j: (0, j)),
            pl.BlockSpec((1, 1), lambda j: (0, 0)),
        ],
        out_shape=[
            jax.ShapeDtypeStruct((1, AD), jnp.float32),
            jax.ShapeDtypeStruct((1, 1), jnp.float32),
        ],
    )(fa, w, ab, fcv, cw, cb)


def kernel(substrate_features, substrate_edge_index, vnr_features,
           actor_conv_W, actor_conv_b, critic_conv_W, critic_conv_b,
           a1_w, a1_b, a2_w, a2_b, a3_w, a3_b,
           c1_w, c1_b, c2_w, c2_b, c3_w, c3_b,
           actor_fc_w, actor_fc_b, critic_fc_w, critic_fc_b):
    x = substrate_features
    ei = substrate_edge_index.astype(jnp.int32)
    row = ei[0]
    col = ei[1]

    wa = jnp.transpose(actor_conv_W, (1, 0, 2)).reshape(D, 9)
    wc = jnp.transpose(critic_conv_W, (1, 0, 2)).reshape(D, 9)
    wcat = jnp.stack([wa, wc])
    z6 = jnp.zeros((6,), jnp.float32)
    bias = jnp.stack([
        jnp.concatenate([actor_conv_b, z6]),
        jnp.concatenate([critic_conv_b, z6]),
    ])[:, None, :]

    yproj = _project(x, wcat, bias)
    yt = jnp.transpose(yproj, (0, 2, 1)).reshape(-1)
    conv = _sc_conv(row, col, yt)
    conv3 = conv.reshape(2, 3, NP)
    ga = jnp.transpose(conv3[0, :, :N], (1, 0)).reshape(-1)
    gc = jnp.transpose(conv3[1, :, :N], (1, 0)).reshape(-1)

    vnr = vnr_features
    va = jnp.concatenate([
        vnr[0] * a1_w[0] + a1_b,
        vnr[1] * a2_w[0] + a2_b,
        vnr[2] * a3_w[0] + a3_b,
    ])
    vc = jnp.concatenate([
        vnr[0] * c1_w[0] + c1_b,
        vnr[1] * c2_w[0] + c2_b,
        vnr[2] * c3_w[0] + c3_b,
    ])
    fa = jnp.concatenate([ga, va])[None]
    fcv = jnp.concatenate([gc, vc])[None]

    logits, values = _fc(
        fa, actor_fc_w, actor_fc_b[None],
        fcv, critic_fc_w, critic_fc_b[None])
    return logits, values
